# Initial kernel scaffold; baseline (speedup 1.0000x reference)
import jax
import jax.numpy as jnp
from jax.experimental import pallas as pl


def kernel(output0, output1, output2, bboxes, labels):
    raise NotImplementedError("write your pallas kernel here")


# speedup vs baseline: 6.1140x; 6.1140x over previous
import functools

import jax
import jax.numpy as jnp
from jax import lax
from jax.experimental import pallas as pl
from jax.experimental.pallas import tpu as pltpu
from jax.experimental.pallas import tpu_sc as plsc

_IMG = 416.0
_B = 16
_K = 20
_C = 28
_NCLS = 20
_SCALES = ((52, 0.0, 128.0), (26, 128.0, 384.0), (13, 384.0, 1200.0))


def _vlog(x):
    bits = lax.bitcast_convert_type(x, jnp.int32)
    e = jnp.right_shift(bits, 23) - 127
    m_bits = jnp.bitwise_or(jnp.bitwise_and(bits, 0x007FFFFF), 0x3F800000)
    m = lax.bitcast_convert_type(m_bits, jnp.float32)
    big = m > (4.0 / 3.0)
    m = jnp.where(big, m * 0.5, m)
    e = jnp.where(big, e + 1, e)
    t = (m - 1.0) / (m + 1.0)
    t2 = t * t
    p = 2.0 * t * (1.0 + t2 * (1.0 / 3.0 + t2 * (0.2 + t2 * (1.0 / 7.0))))
    return p + e.astype(jnp.float32) * 0.6931471805599453


def _smooth_l1(d):
    ad = jnp.abs(d)
    return jnp.where(ad < 1.0, 0.5 * ad * ad, ad - 0.5)


def _group(table, S, lo, hi, b, g, bbox_hbm, lab_hbm, out_hbm,
           bbox_v, lab_v, idx_v, rows_v, win_v, part_v, sem):
    iota = lax.iota(jnp.int32, 16)
    pltpu.sync_copy(bbox_hbm.at[b], bbox_v)
    pltpu.sync_copy(lab_hbm.at[b], lab_v)

    base_ch = (b * _C) * (S * S)
    halves = []
    for hh in (0, 1):
        off = 16 * hh
        sl = pl.ds(off, 16)
        xs = bbox_v[0, sl]
        ys = bbox_v[1, sl]
        us = bbox_v[2, sl]
        vs = bbox_v[3, sl]
        sf = bbox_v[4, sl]
        ps = bbox_v[5, sl]
        d2 = (4.0 * _IMG * _IMG) * (us * us + vs * vs)
        valid = (iota + off) < _K
        m = (d2 >= lo * lo) & (d2 <= hi * hi) & valid
        gx = xs * float(S)
        gy = ys * float(S)
        xi = gx.astype(jnp.int32)
        yi = gy.astype(jnp.int32)
        xf = gx - xi.astype(jnp.float32)
        yf = gy - yi.astype(jnp.float32)
        xi = jnp.minimum(jnp.maximum(xi, 0), S - 1)
        yi = jnp.minimum(jnp.maximum(yi, 0), S - 1)
        lin = yi * S + xi
        for c in range(_C):
            idx_v[c, sl] = lin + (base_ch + c * (S * S))
        halves.append((m, lin, xf, yf, us, vs, ps, sf, m.astype(jnp.int32)))

    descs = [
        pltpu.async_copy(table.at[idx_v.at[c]], rows_v.at[c], sem)
        for c in range(_C)
    ]

    for hh in (0, 1):
        off = 16 * hh
        m, lin = halves[hh][0], halves[hh][1]
        plsc.store_scatter(win_v, [lin], iota + off, mask=m)
    winners = []
    for hh in (0, 1):
        off = 16 * hh
        m, lin = halves[hh][0], halves[hh][1]
        t = plsc.load_gather(win_v, [lin])
        winners.append(m & (t == iota + off))

    for d in descs:
        d.wait()

    acc = [0.0] * 6
    for hh in (0, 1):
        off = 16 * hh
        sl = pl.ds(off, 16)
        m, _lin, xf, yf, us, vs, ps, sf, _mi = halves[hh]
        win = winners[hh]
        winf = jnp.where(win, 1.0, 0.0).astype(jnp.float32)

        bb = _smooth_l1(rows_v[0, sl] - xf)
        bb = bb + _smooth_l1(rows_v[1, sl] - yf)
        bb = bb + _smooth_l1(rows_v[2, sl] - us)
        bb = bb + _smooth_l1(rows_v[3, sl] - vs)
        bb = bb + _smooth_l1(rows_v[4, sl] - ps)

        l5 = rows_v[5, sl]
        l6 = rows_v[6, sl]
        mx2 = jnp.maximum(l5, l6)
        se2 = jnp.exp(l5 - mx2) + jnp.exp(l6 - mx2)
        si = sf.astype(jnp.int32)
        lsel = jnp.where(si == 0, l5, l6)
        ce2 = _vlog(se2) + mx2 - lsel

        s7 = rows_v[7, sl]
        sq = s7 * s7
        sp = (s7 - 1.0) * (s7 - 1.0)

        mx20 = rows_v[8, sl]
        for c in range(9, 8 + _NCLS):
            mx20 = jnp.maximum(mx20, rows_v[c, sl])
        se20 = jnp.exp(rows_v[8, sl] - mx20)
        for c in range(9, 8 + _NCLS):
            se20 = se20 + jnp.exp(rows_v[c, sl] - mx20)
        lt = plsc.load_gather(rows_v, [lab_v[sl] + 8, iota + off])
        ce20 = _vlog(se20) + mx20 - lt

        acc[0] = acc[0] + jnp.sum(jnp.where(m, 1.0, 0.0).astype(jnp.float32))
        acc[1] = acc[1] + jnp.sum(sq * winf)
        acc[2] = acc[2] + jnp.sum(sp * winf)
        acc[3] = acc[3] + jnp.sum(bb * winf)
        acc[4] = acc[4] + jnp.sum(ce2 * winf)
        acc[5] = acc[5] + jnp.sum(ce20 * winf)

    pv = jnp.zeros((16,), jnp.float32)
    for i in range(6):
        pv = jnp.where(iota == i, acc[i], pv)
    part_v[...] = pv
    pltpu.sync_copy(part_v, out_hbm.at[g])


@functools.partial(
    pl.kernel,
    mesh=plsc.VectorSubcoreMesh(core_axis_name="c", subcore_axis_name="s"),
    out_type=jax.ShapeDtypeStruct((48, 16), jnp.float32),
    compiler_params=pltpu.CompilerParams(needs_layout_passes=False),
    scratch_types=[
        pltpu.VMEM((6, 32), jnp.float32),
        pltpu.VMEM((32,), jnp.int32),
        pltpu.VMEM((_C, 32), jnp.int32),
        pltpu.VMEM((_C, 32), jnp.float32),
        pltpu.VMEM((52 * 52,), jnp.int32),
        pltpu.VMEM((16,), jnp.float32),
        pltpu.SemaphoreType.DMA,
    ],
)
def _sc_parts(t0, t1, t2, bbox_hbm, lab_hbm, out_hbm,
              bbox_v, lab_v, idx_v, rows_v, win_v, part_v, sem):
    wid = lax.axis_index("c") * 16 + lax.axis_index("s")
    scr = (bbox_hbm, lab_hbm, out_hbm,
           bbox_v, lab_v, idx_v, rows_v, win_v, part_v, sem)

    @pl.when(wid < 16)
    def _():
        s, lo, hi = _SCALES[0]
        _group(t0, s, lo, hi, wid, wid, *scr)

    @pl.when(wid >= 16)
    def _():
        b = wid - 16
        s, lo, hi = _SCALES[1]
        _group(t1, s, lo, hi, b, b + 16, *scr)
        s, lo, hi = _SCALES[2]
        _group(t2, s, lo, hi, b, b + 32, *scr)


def _combine_body(s0_ref, s1_ref, s2_ref, parts_ref, out_ref):
    p = parts_ref[...]
    total = jnp.float32(0.0)
    for l, (s_ref, (S, _lo, _hi)) in enumerate(
            zip((s0_ref, s1_ref, s2_ref), _SCALES)):
        sq_all = jnp.sum(s_ref[...] * s_ref[...])
        blk = p[l * 16:(l + 1) * 16, :]
        nm = jnp.sum(blk[:, 0:1])
        sqp = jnp.sum(blk[:, 1:2])
        slp = jnp.sum(blk[:, 2:3])
        bbv = jnp.sum(blk[:, 3:4])
        bbs = jnp.sum(blk[:, 4:5])
        lbl = jnp.sum(blk[:, 5:6])
        pix_p = nm
        pix_n = float(_B * S * S) - nm
        sln = sq_all - sqp
        sln = jnp.where(pix_n > 0, sln / pix_n, sln)
        slp = jnp.where(pix_p > 0, slp / pix_p, slp)
        bbs = jnp.where(pix_p > 0, bbs / pix_p, bbs)
        bbv = jnp.where(pix_p > 0, bbv / pix_p, bbv)
        lbl = jnp.where(pix_p > 0, lbl / pix_p, lbl)
        total = total + (0.25 * sln + 1.75 * slp + 0.8 * bbs
                         + 1.2 * bbv + 2.0 * lbl)
    out_ref[...] = jnp.reshape(total, (1, 1))


def kernel(output0, output1, output2, bboxes, labels):
    bbox_t = jnp.pad(bboxes.transpose(0, 2, 1), ((0, 0), (0, 0), (0, 12)))
    lab_p = jnp.pad(labels, ((0, 0), (0, 12)))
    parts = _sc_parts(
        output0.reshape(-1), output1.reshape(-1), output2.reshape(-1),
        bbox_t, lab_p)
    out = pl.pallas_call(
        _combine_body,
        out_shape=jax.ShapeDtypeStruct((1, 1), jnp.float32),
    )(output0[:, 7], output1[:, 7], output2[:, 7], parts)
    return out.reshape((1,))


# speedup vs baseline: 6.1376x; 1.0039x over previous
import functools

import jax
import jax.numpy as jnp
from jax import lax
from jax.experimental import pallas as pl
from jax.experimental.pallas import tpu as pltpu
from jax.experimental.pallas import tpu_sc as plsc

_IMG = 416.0
_B = 16
_K = 20
_C = 28
_NCLS = 20
_SCALES = ((52, 0.0, 128.0), (26, 128.0, 384.0), (13, 384.0, 1200.0))


def _vlog(x):
    bits = lax.bitcast_convert_type(x, jnp.int32)
    e = jnp.right_shift(bits, 23) - 127
    m_bits = jnp.bitwise_or(jnp.bitwise_and(bits, 0x007FFFFF), 0x3F800000)
    m = lax.bitcast_convert_type(m_bits, jnp.float32)
    big = m > (4.0 / 3.0)
    m = jnp.where(big, m * 0.5, m)
    e = jnp.where(big, e + 1, e)
    t = (m - 1.0) / (m + 1.0)
    t2 = t * t
    p = 2.0 * t * (1.0 + t2 * (1.0 / 3.0 + t2 * (0.2 + t2 * (1.0 / 7.0))))
    return p + e.astype(jnp.float32) * 0.6931471805599453


def _smooth_l1(d):
    ad = jnp.abs(d)
    return jnp.where(ad < 1.0, 0.5 * ad * ad, ad - 0.5)


def _group(table, S, lo, hi, b, g, bbox_hbm, lab_hbm, out_hbm,
           bbox_v, lab_v, idx_v, rows_v, win_v, part_v, sem):
    iota = lax.iota(jnp.int32, 16)
    pltpu.sync_copy(bbox_hbm.at[b], bbox_v)
    pltpu.sync_copy(lab_hbm.at[b], lab_v)

    base_ch = (b * _C) * (S * S)
    halves = []
    for hh in (0, 1):
        off = 16 * hh
        sl = pl.ds(off, 16)
        xs = bbox_v[0, sl]
        ys = bbox_v[1, sl]
        us = bbox_v[2, sl]
        vs = bbox_v[3, sl]
        sf = bbox_v[4, sl]
        ps = bbox_v[5, sl]
        d2 = (4.0 * _IMG * _IMG) * (us * us + vs * vs)
        valid = (iota + off) < _K
        m = (d2 >= lo * lo) & (d2 <= hi * hi) & valid
        gx = xs * float(S)
        gy = ys * float(S)
        xi = gx.astype(jnp.int32)
        yi = gy.astype(jnp.int32)
        xf = gx - xi.astype(jnp.float32)
        yf = gy - yi.astype(jnp.float32)
        xi = jnp.minimum(jnp.maximum(xi, 0), S - 1)
        yi = jnp.minimum(jnp.maximum(yi, 0), S - 1)
        lin = yi * S + xi
        for c in range(_C):
            idx_v[c, sl] = lin + (base_ch + c * (S * S))
        halves.append((m, lin, xf, yf, us, vs, ps, sf, m.astype(jnp.int32)))

    flat = table
    descs = [
        pltpu.async_copy(flat.at[idx_v.at[c]], rows_v.at[c], sem)
        for c in range(_C)
    ]

    for hh in (0, 1):
        off = 16 * hh
        m, lin = halves[hh][0], halves[hh][1]
        plsc.store_scatter(win_v, [lin], iota + off, mask=m)
    winners = []
    for hh in (0, 1):
        off = 16 * hh
        m, lin = halves[hh][0], halves[hh][1]
        t = plsc.load_gather(win_v, [lin])
        winners.append(m & (t == iota + off))

    for d in descs:
        d.wait()

    acc = [0.0] * 6
    for hh in (0, 1):
        off = 16 * hh
        sl = pl.ds(off, 16)
        m, _lin, xf, yf, us, vs, ps, sf, _mi = halves[hh]
        win = winners[hh]
        winf = jnp.where(win, 1.0, 0.0).astype(jnp.float32)

        bb = _smooth_l1(rows_v[0, sl] - xf)
        bb = bb + _smooth_l1(rows_v[1, sl] - yf)
        bb = bb + _smooth_l1(rows_v[2, sl] - us)
        bb = bb + _smooth_l1(rows_v[3, sl] - vs)
        bb = bb + _smooth_l1(rows_v[4, sl] - ps)

        l5 = rows_v[5, sl]
        l6 = rows_v[6, sl]
        mx2 = jnp.maximum(l5, l6)
        se2 = jnp.exp(l5 - mx2) + jnp.exp(l6 - mx2)
        si = sf.astype(jnp.int32)
        lsel = jnp.where(si == 0, l5, l6)
        ce2 = _vlog(se2) + mx2 - lsel

        s7 = rows_v[7, sl]
        sq = s7 * s7
        sp = (s7 - 1.0) * (s7 - 1.0)

        mx20 = rows_v[8, sl]
        for c in range(9, 8 + _NCLS):
            mx20 = jnp.maximum(mx20, rows_v[c, sl])
        se20 = jnp.exp(rows_v[8, sl] - mx20)
        for c in range(9, 8 + _NCLS):
            se20 = se20 + jnp.exp(rows_v[c, sl] - mx20)
        lt = plsc.load_gather(rows_v, [lab_v[sl] + 8, iota + off])
        ce20 = _vlog(se20) + mx20 - lt

        acc[0] = acc[0] + jnp.sum(jnp.where(m, 1.0, 0.0).astype(jnp.float32))
        acc[1] = acc[1] + jnp.sum(sq * winf)
        acc[2] = acc[2] + jnp.sum(sp * winf)
        acc[3] = acc[3] + jnp.sum(bb * winf)
        acc[4] = acc[4] + jnp.sum(ce2 * winf)
        acc[5] = acc[5] + jnp.sum(ce20 * winf)

    pv = jnp.zeros((16,), jnp.float32)
    for i in range(6):
        pv = jnp.where(iota == i, acc[i], pv)
    part_v[...] = pv
    pltpu.sync_copy(part_v, out_hbm.at[g])


@functools.partial(
    pl.kernel,
    mesh=plsc.VectorSubcoreMesh(core_axis_name="c", subcore_axis_name="s"),
    out_type=jax.ShapeDtypeStruct((48, 16), jnp.float32),
    compiler_params=pltpu.CompilerParams(needs_layout_passes=False),
    scratch_types=[
        pltpu.VMEM((6, 32), jnp.float32),
        pltpu.VMEM((32,), jnp.int32),
        pltpu.VMEM((_C, 32), jnp.int32),
        pltpu.VMEM((_C, 32), jnp.float32),
        pltpu.VMEM((52 * 52,), jnp.int32),
        pltpu.VMEM((16,), jnp.float32),
        pltpu.SemaphoreType.DMA,
    ],
)
def _sc_parts(t0, t1, t2, bbox_hbm, lab_hbm, out_hbm,
              bbox_v, lab_v, idx_v, rows_v, win_v, part_v, sem):
    wid = lax.axis_index("c") * 16 + lax.axis_index("s")
    scr = (bbox_hbm, lab_hbm, out_hbm,
           bbox_v, lab_v, idx_v, rows_v, win_v, part_v, sem)

    @pl.when(wid < 16)
    def _():
        s, lo, hi = _SCALES[0]
        _group(t0, s, lo, hi, wid, wid, *scr)

    @pl.when(wid >= 16)
    def _():
        b = wid - 16
        s, lo, hi = _SCALES[1]
        _group(t1, s, lo, hi, b, b + 16, *scr)
        s, lo, hi = _SCALES[2]
        _group(t2, s, lo, hi, b, b + 32, *scr)


def _combine_body(s0_ref, s1_ref, s2_ref, parts_ref, out_ref):
    p = parts_ref[...]
    total = jnp.float32(0.0)
    for l, (s_ref, (S, _lo, _hi)) in enumerate(
            zip((s0_ref, s1_ref, s2_ref), _SCALES)):
        sq_all = jnp.sum(s_ref[...] * s_ref[...])
        blk = p[l * 16:(l + 1) * 16, :]
        nm = jnp.sum(blk[:, 0:1])
        sqp = jnp.sum(blk[:, 1:2])
        slp = jnp.sum(blk[:, 2:3])
        bbv = jnp.sum(blk[:, 3:4])
        bbs = jnp.sum(blk[:, 4:5])
        lbl = jnp.sum(blk[:, 5:6])
        pix_p = nm
        pix_n = float(_B * S * S) - nm
        sln = sq_all - sqp
        sln = jnp.where(pix_n > 0, sln / pix_n, sln)
        slp = jnp.where(pix_p > 0, slp / pix_p, slp)
        bbs = jnp.where(pix_p > 0, bbs / pix_p, bbs)
        bbv = jnp.where(pix_p > 0, bbv / pix_p, bbv)
        lbl = jnp.where(pix_p > 0, lbl / pix_p, lbl)
        total = total + (0.25 * sln + 1.75 * slp + 0.8 * bbs
                         + 1.2 * bbv + 2.0 * lbl)
    out_ref[...] = jnp.reshape(total, (1, 1))


def kernel(output0, output1, output2, bboxes, labels):
    bbox_t = jnp.pad(bboxes.transpose(0, 2, 1), ((0, 0), (0, 0), (0, 12)))
    lab_p = jnp.pad(labels, ((0, 0), (0, 12)))
    parts = _sc_parts(output0.reshape(-1), output1.reshape(-1),
                      output2.reshape(-1), bbox_t, lab_p)
    out = pl.pallas_call(
        _combine_body,
        out_shape=jax.ShapeDtypeStruct((1, 1), jnp.float32),
    )(output0[:, 7], output1[:, 7], output2[:, 7], parts)
    return out.reshape((1,))


# speedup vs baseline: 6.8636x; 1.1183x over previous
import functools

import jax
import jax.numpy as jnp
from jax import lax
from jax.experimental import pallas as pl
from jax.experimental.pallas import tpu as pltpu
from jax.experimental.pallas import tpu_sc as plsc

_IMG = 416.0
_B = 16
_K = 20
_C = 28
_NCLS = 20
_SCALES = ((52, 0.0, 128.0), (26, 128.0, 384.0), (13, 384.0, 1200.0))


def _vlog(x):
    bits = lax.bitcast_convert_type(x, jnp.int32)
    e = jnp.right_shift(bits, 23) - 127
    m_bits = jnp.bitwise_or(jnp.bitwise_and(bits, 0x007FFFFF), 0x3F800000)
    m = lax.bitcast_convert_type(m_bits, jnp.float32)
    big = m > (4.0 / 3.0)
    m = jnp.where(big, m * 0.5, m)
    e = jnp.where(big, e + 1, e)
    t = (m - 1.0) / (m + 1.0)
    t2 = t * t
    p = 2.0 * t * (1.0 + t2 * (1.0 / 3.0 + t2 * (0.2 + t2 * (1.0 / 7.0))))
    return p + e.astype(jnp.float32) * 0.6931471805599453


def _smooth_l1(d):
    ad = jnp.abs(d)
    return jnp.where(ad < 1.0, 0.5 * ad * ad, ad - 0.5)


def _group(table, S, lo, hi, b, g, bbox_hbm, lab_hbm, out_hbm,
           bbox_v, lab_v, lidx_v, idx_v, rows_v, win_v, score_v, part_v,
           sem, sem2, sem3):
    iota = lax.iota(jnp.int32, 16)
    pltpu.sync_copy(bbox_hbm.at[pl.ds(b * (_K * 6), _K * 6)], bbox_v)

    for hh in (0, 1):
        lidx_v[pl.ds(16 * hh, 16)] = b * _K + jnp.minimum(
            iota + 16 * hh, _K - 1)
    ldesc = pltpu.async_copy(lab_hbm.at[lidx_v], lab_v, sem3)

    start = (b * _C + 7) * (S * S)
    if (S * S) % 8 == 0:
        s_sh = 0
        sdesc = pltpu.async_copy(
            table.at[pl.ds(start, S * S)], score_v.at[pl.ds(0, S * S)], sem2)
    else:
        start_al = jnp.right_shift(start, 3) * 8
        s_ln = ((S * S + 15) // 8) * 8
        s_sh = start - start_al
        sdesc = pltpu.async_copy(
            table.at[pl.ds(start_al, s_ln)], score_v.at[pl.ds(0, s_ln)], sem2)

    base_ch = (b * _C) * (S * S)
    halves = []
    for hh in (0, 1):
        off = 16 * hh
        bidx = jnp.minimum(iota + off, _K - 1) * 6
        xs = plsc.load_gather(bbox_v, [bidx])
        ys = plsc.load_gather(bbox_v, [bidx + 1])
        us = plsc.load_gather(bbox_v, [bidx + 2])
        vs = plsc.load_gather(bbox_v, [bidx + 3])
        sf = plsc.load_gather(bbox_v, [bidx + 4])
        ps = plsc.load_gather(bbox_v, [bidx + 5])
        d2 = (4.0 * _IMG * _IMG) * (us * us + vs * vs)
        valid = (iota + off) < _K
        m = (d2 >= lo * lo) & (d2 <= hi * hi) & valid
        gx = xs * float(S)
        gy = ys * float(S)
        xi = gx.astype(jnp.int32)
        yi = gy.astype(jnp.int32)
        xf = gx - xi.astype(jnp.float32)
        yf = gy - yi.astype(jnp.float32)
        xi = jnp.minimum(jnp.maximum(xi, 0), S - 1)
        yi = jnp.minimum(jnp.maximum(yi, 0), S - 1)
        lin = yi * S + xi
        for c in range(_C):
            idx_v[pl.ds(c * 32 + off, 16)] = lin + (base_ch + c * (S * S))
        halves.append((m, lin, xf, yf, us, vs, ps, sf))

    desc = pltpu.async_copy(table.at[idx_v], rows_v, sem)

    for hh in (0, 1):
        off = 16 * hh
        m, lin = halves[hh][0], halves[hh][1]
        plsc.store_scatter(win_v, [lin], iota + off, mask=m)
    winners = []
    for hh in (0, 1):
        off = 16 * hh
        m, lin = halves[hh][0], halves[hh][1]
        t = plsc.load_gather(win_v, [lin])
        winners.append(m & (t == iota + off))

    desc.wait()

    acc = [0.0] * 6
    for hh in (0, 1):
        off = 16 * hh
        m, _lin, xf, yf, us, vs, ps, sf = halves[hh]
        win = winners[hh]
        winf = jnp.where(win, 1.0, 0.0).astype(jnp.float32)

        def row(c, off=off):
            return rows_v[pl.ds(c * 32 + off, 16)]

        bb = _smooth_l1(row(0) - xf)
        bb = bb + _smooth_l1(row(1) - yf)
        bb = bb + _smooth_l1(row(2) - us)
        bb = bb + _smooth_l1(row(3) - vs)
        bb = bb + _smooth_l1(row(4) - ps)

        l5 = row(5)
        l6 = row(6)
        mx2 = jnp.maximum(l5, l6)
        se2 = jnp.exp(l5 - mx2) + jnp.exp(l6 - mx2)
        si = sf.astype(jnp.int32)
        lsel = jnp.where(si == 0, l5, l6)
        ce2 = _vlog(se2) + mx2 - lsel

        s7 = row(7)
        sq = s7 * s7
        sp = (s7 - 1.0) * (s7 - 1.0)

        mx20 = row(8)
        for c in range(9, 8 + _NCLS):
            mx20 = jnp.maximum(mx20, row(c))
        se20 = jnp.exp(row(8) - mx20)
        for c in range(9, 8 + _NCLS):
            se20 = se20 + jnp.exp(row(c) - mx20)
        if hh == 0:
            ldesc.wait()
        lab = jnp.minimum(jnp.maximum(lab_v[pl.ds(off, 16)], 0), _NCLS - 1)
        lt = plsc.load_gather(rows_v, [(lab + 8) * 32 + iota + off])
        ce20 = _vlog(se20) + mx20 - lt

        acc[0] = acc[0] + jnp.sum(jnp.where(m, 1.0, 0.0).astype(jnp.float32))
        acc[1] = acc[1] + jnp.sum(sq * winf)
        acc[2] = acc[2] + jnp.sum(sp * winf)
        acc[3] = acc[3] + jnp.sum(bb * winf)
        acc[4] = acc[4] + jnp.sum(ce2 * winf)
        acc[5] = acc[5] + jnp.sum(ce20 * winf)

    sdesc.wait()
    sacc = jnp.zeros((16,), jnp.float32)
    n_full = S * S // 16
    rem = S * S - n_full * 16
    if (S * S) % 8 == 0:
        for i in range(n_full):
            v = score_v[pl.ds(16 * i, 16)]
            sacc = sacc + v * v
        if rem:
            v = score_v[pl.ds(S * S - rem, 16)]
            v = jnp.where(iota < rem, v, 0.0)
            sacc = sacc + v * v
    else:
        for i in range(n_full):
            v = plsc.load_gather(score_v, [s_sh + 16 * i + iota])
            sacc = sacc + v * v
        if rem:
            v = plsc.load_gather(
                score_v, [s_sh + jnp.minimum(n_full * 16 + iota, S * S - 1)])
            v = jnp.where(iota < rem, v, 0.0)
            sacc = sacc + v * v
    sq_all = jnp.sum(sacc)

    pv = jnp.zeros((16,), jnp.float32)
    for i in range(6):
        pv = jnp.where(iota == i, acc[i], pv)
    pv = jnp.where(iota == 6, sq_all, pv)
    part_v[...] = pv
    pltpu.sync_copy(part_v, out_hbm.at[g])


@functools.partial(
    pl.kernel,
    mesh=plsc.VectorSubcoreMesh(core_axis_name="c", subcore_axis_name="s"),
    out_type=jax.ShapeDtypeStruct((48, 16), jnp.float32),
    compiler_params=pltpu.CompilerParams(needs_layout_passes=False),
    scratch_types=[
        pltpu.VMEM((_K * 6,), jnp.float32),
        pltpu.VMEM((32,), jnp.int32),
        pltpu.VMEM((32,), jnp.int32),
        pltpu.VMEM((_C * 32,), jnp.int32),
        pltpu.VMEM((_C * 32,), jnp.float32),
        pltpu.VMEM((52 * 52,), jnp.int32),
        pltpu.VMEM((52 * 52,), jnp.float32),
        pltpu.VMEM((16,), jnp.float32),
        pltpu.SemaphoreType.DMA,
        pltpu.SemaphoreType.DMA,
        pltpu.SemaphoreType.DMA,
    ],
)
def _sc_parts(t0, t1, t2, bbox_hbm, lab_hbm, out_hbm,
              bbox_v, lab_v, lidx_v, idx_v, rows_v, win_v, score_v, part_v,
              sem, sem2, sem3):
    wid = lax.axis_index("c") * 16 + lax.axis_index("s")
    scr = (bbox_hbm, lab_hbm, out_hbm,
           bbox_v, lab_v, lidx_v, idx_v, rows_v, win_v, score_v, part_v,
           sem, sem2, sem3)

    @pl.when(wid < 16)
    def _():
        s, lo, hi = _SCALES[0]
        _group(t0, s, lo, hi, wid, wid, *scr)

    @pl.when(wid >= 16)
    def _():
        b = wid - 16
        s, lo, hi = _SCALES[1]
        _group(t1, s, lo, hi, b, b + 16, *scr)
        s, lo, hi = _SCALES[2]
        _group(t2, s, lo, hi, b, b + 32, *scr)


def _combine_body(parts_ref, out_ref):
    p = parts_ref[...]
    total = jnp.float32(0.0)
    for l, (S, _lo, _hi) in enumerate(_SCALES):
        blk = p[l * 16:(l + 1) * 16, :]
        nm = jnp.sum(blk[:, 0:1])
        sqp = jnp.sum(blk[:, 1:2])
        slp = jnp.sum(blk[:, 2:3])
        bbv = jnp.sum(blk[:, 3:4])
        bbs = jnp.sum(blk[:, 4:5])
        lbl = jnp.sum(blk[:, 5:6])
        sq_all = jnp.sum(blk[:, 6:7])
        pix_p = nm
        pix_n = float(_B * S * S) - nm
        sln = sq_all - sqp
        sln = jnp.where(pix_n > 0, sln / pix_n, sln)
        slp = jnp.where(pix_p > 0, slp / pix_p, slp)
        bbs = jnp.where(pix_p > 0, bbs / pix_p, bbs)
        bbv = jnp.where(pix_p > 0, bbv / pix_p, bbv)
        lbl = jnp.where(pix_p > 0, lbl / pix_p, lbl)
        total = total + (0.25 * sln + 1.75 * slp + 0.8 * bbs
                         + 1.2 * bbv + 2.0 * lbl)
    out_ref[...] = jnp.reshape(total, (1, 1))


def kernel(output0, output1, output2, bboxes, labels):
    parts = _sc_parts(output0.reshape(-1), output1.reshape(-1),
                      output2.reshape(-1), bboxes.reshape(-1),
                      labels.reshape(-1).astype(jnp.int32))
    out = pl.pallas_call(
        _combine_body,
        out_shape=jax.ShapeDtypeStruct((1, 1), jnp.float32),
    )(parts)
    return out.reshape((1,))
